# Initial kernel scaffold; baseline (speedup 1.0000x reference)
import jax
import jax.numpy as jnp
from jax.experimental import pallas as pl


def kernel(x, W, b):
    raise NotImplementedError("write your pallas kernel here")


# speedup vs baseline: 16.5923x; 16.5923x over previous
import functools

import jax
import jax.numpy as jnp
from jax.experimental import pallas as pl

_B = 1024
_D = 1024
_N_POP = 256
_T = 256
_TAU = 10.0
_SCALE = _T * _TAU / (_TAU + 1.0)

_BB = 16


def _onehot_block(x_ref, w_ref, b_ref, out_ref):
    z = jnp.dot(x_ref[...], w_ref[...], preferred_element_type=jnp.float32)
    intensity = jax.nn.sigmoid(z + b_ref[...])
    st = jnp.clip(((1.0 - intensity) * _SCALE).astype(jnp.int32), 0, _T - 1)
    t_iota = jax.lax.broadcasted_iota(jnp.int32, (_BB, _T, _N_POP), 1)
    out_ref[...] = (t_iota == st[:, None, :]).astype(jnp.float32)


@functools.partial(jax.jit)
def kernel(x, W, b):
    wt = W.T
    b2 = b.reshape(1, _N_POP)
    return pl.pallas_call(
        _onehot_block,
        grid=(_B // _BB,),
        in_specs=[
            pl.BlockSpec((_BB, _D), lambda i: (i, 0)),
            pl.BlockSpec((_D, _N_POP), lambda i: (0, 0)),
            pl.BlockSpec((1, _N_POP), lambda i: (0, 0)),
        ],
        out_specs=pl.BlockSpec((_BB, _T, _N_POP), lambda i: (i, 0, 0)),
        out_shape=jax.ShapeDtypeStruct((_B, _T, _N_POP), jnp.float32),
    )(x, wt, b2)


# speedup vs baseline: 16.8970x; 1.0184x over previous
import functools

import jax
import jax.numpy as jnp
from jax.experimental import pallas as pl

_B = 1024
_D = 1024
_N_POP = 256
_T = 256
_TAU = 10.0
_SCALE = _T * _TAU / (_TAU + 1.0)

_BB = 64


def _onehot_block(x_ref, w_ref, b_ref, out_ref):
    z = jnp.dot(x_ref[...], w_ref[...], preferred_element_type=jnp.float32)
    intensity = jax.nn.sigmoid(z + b_ref[...])
    st = jnp.clip(((1.0 - intensity) * _SCALE).astype(jnp.int32), 0, _T - 1)
    t_iota = jax.lax.broadcasted_iota(jnp.int32, (_BB, _T, _N_POP), 1)
    out_ref[...] = (t_iota == st[:, None, :]).astype(jnp.float32)


@functools.partial(jax.jit)
def kernel(x, W, b):
    wt = W.T
    b2 = b.reshape(1, _N_POP)
    return pl.pallas_call(
        _onehot_block,
        grid=(_B // _BB,),
        in_specs=[
            pl.BlockSpec((_BB, _D), lambda i: (i, 0)),
            pl.BlockSpec((_D, _N_POP), lambda i: (0, 0)),
            pl.BlockSpec((1, _N_POP), lambda i: (0, 0)),
        ],
        out_specs=pl.BlockSpec((_BB, _T, _N_POP), lambda i: (i, 0, 0)),
        out_shape=jax.ShapeDtypeStruct((_B, _T, _N_POP), jnp.float32),
    )(x, wt, b2)


# speedup vs baseline: 17.2855x; 1.0230x over previous
import functools

import jax
import jax.numpy as jnp
from jax.experimental import pallas as pl

_B = 1024
_D = 1024
_N_POP = 256
_T = 256
_TAU = 10.0
_SCALE = _T * _TAU / (_TAU + 1.0)

_BB = 32


def _onehot_block(x_ref, w_ref, b_ref, out_ref):
    z = jnp.dot(x_ref[...], w_ref[...], preferred_element_type=jnp.float32)
    intensity = jax.nn.sigmoid(z + b_ref[...])
    st = jnp.clip(((1.0 - intensity) * _SCALE).astype(jnp.int32), 0, _T - 1)
    t_iota = jax.lax.broadcasted_iota(jnp.int32, (_BB, _T, _N_POP), 1)
    out_ref[...] = (t_iota == st[:, None, :]).astype(jnp.float32)


@functools.partial(jax.jit)
def kernel(x, W, b):
    wt = W.T
    b2 = b.reshape(1, _N_POP)
    return pl.pallas_call(
        _onehot_block,
        grid=(_B // _BB,),
        in_specs=[
            pl.BlockSpec((_BB, _D), lambda i: (i, 0)),
            pl.BlockSpec((_D, _N_POP), lambda i: (0, 0)),
            pl.BlockSpec((1, _N_POP), lambda i: (0, 0)),
        ],
        out_specs=pl.BlockSpec((_BB, _T, _N_POP), lambda i: (i, 0, 0)),
        out_shape=jax.ShapeDtypeStruct((_B, _T, _N_POP), jnp.float32),
    )(x, wt, b2)
